# baseline (speedup 1.0000x reference)
import functools
import jax
import jax.numpy as jnp
from jax import lax
from jax.experimental import pallas as pl
from jax.experimental.pallas import tpu as pltpu
from jax.experimental.pallas import tpu_sc as plsc

N = 50000
D = 512
G = 256
B = 2000
NB = N // B

NW = 32
SEG_PER_W = G // NW
RB = 16
ACC = SEG_PER_W * D

_NEG = -1e30


def _gate_kernel(feat, seg, wg, bg, gate, m_out, cnt_out):
    i = pl.program_id(0)

    @pl.when(i == 0)
    def _():
        m_out[...] = jnp.full_like(m_out, _NEG)
        cnt_out[...] = jnp.zeros_like(cnt_out)

    x = feat[...]
    g = jnp.dot(x, wg[...], preferred_element_type=jnp.float32) + bg[0, 0]
    gate[...] = g
    s = seg[0, 0, :]
    ids = lax.broadcasted_iota(jnp.int32, (B, G), 1)
    oh = s[:, None] == ids
    lm = jnp.max(jnp.where(oh, g, _NEG), axis=0)
    m_out[0, :] = jnp.maximum(m_out[0, :], lm)
    cnt_out[0, :] = cnt_out[0, :] + jnp.sum(oh.astype(jnp.float32), axis=0)


def _pexp_kernel(gate, seg, m, p_out, den_out):
    i = pl.program_id(0)

    @pl.when(i == 0)
    def _():
        den_out[...] = jnp.zeros_like(den_out)

    g = gate[...]
    s = seg[0, 0, :]
    ids = lax.broadcasted_iota(jnp.int32, (B, G), 1)
    oh = s[:, None] == ids
    m_sel = jnp.sum(jnp.where(oh, m[0, :][None, :], 0.0), axis=1)
    pv = jnp.exp(g[:, 0] - m_sel)
    p_out[...] = pv[:, None]
    den_out[0, :] = den_out[0, :] + jnp.sum(jnp.where(oh, pv[:, None], 0.0), axis=0)


def _pool_kernel(feat, seg, p, pooled):
    i = pl.program_id(0)

    @pl.when(i == 0)
    def _():
        pooled[...] = jnp.zeros_like(pooled)

    x = feat[...]
    w = x * p[...]
    s = seg[0, 0, :]
    ids = lax.broadcasted_iota(jnp.int32, (B, G), 1)
    oh = (s[:, None] == ids).astype(jnp.float32)
    pooled[...] = pooled[...] + lax.dot_general(
        oh, w, dimension_numbers=(((0,), (0,)), ((), ())),
        preferred_element_type=jnp.float32)


def _sc_pool_kernel(feat_hbm, p_hbm, seg_hbm, starts_hbm, out_hbm,
                    sv, fbuf, pbuf, sbuf, acc):
    c = lax.axis_index("c")
    s = lax.axis_index("s")
    w = s * 2 + c
    lo8 = w * SEG_PER_W
    pltpu.sync_copy(starts_hbm.at[pl.ds(lo8, 16)], sv)
    svv = sv[...]
    lo = svv[0]
    hi = svv[SEG_PER_W]
    b0 = lo // RB
    b1 = (hi + RB - 1) // RB

    zeros16 = jnp.zeros((16,), jnp.float32)
    for k in range(ACC // 16):
        acc[pl.ds(16 * k, 16)] = zeros16

    iotas = [16 * j + lax.iota(jnp.int32, 16) for j in range(D // 16)]

    def body(i, carry):
        pltpu.sync_copy(feat_hbm.at[pl.ds(i * RB, RB), :], fbuf)
        pltpu.sync_copy(p_hbm.at[pl.ds(i * RB, RB)], pbuf)
        pltpu.sync_copy(seg_hbm.at[pl.ds(i * RB, RB)], sbuf)
        pvec = pbuf[...]
        svec = sbuf[...]
        for r in range(RB):
            s_r = svec[r]
            p_r = pvec[r]
            ok = (s_r >= lo8) & (s_r < lo8 + SEG_PER_W)
            msk = jnp.broadcast_to(ok, (16,))
            base = jnp.clip(s_r - lo8, 0, SEG_PER_W - 1) * D
            for j in range(D // 16):
                v = fbuf[r, pl.ds(16 * j, 16)] * p_r
                plsc.addupdate_scatter(acc, [base + iotas[j]], v, mask=msk)
        return carry

    lax.fori_loop(b0, b1, body, 0)
    pltpu.sync_copy(acc, out_hbm.at[pl.ds(w * ACC, ACC)])


def _final_kernel(pooled, den, cnt, wn, bn, out):
    d = den[0, :][:, None]
    msk = cnt[0, :][:, None] > 0.5
    inv = jnp.where(d > 0, 1.0 / jnp.where(d > 0, d, 1.0), 0.0)
    pn = pooled[...] * inv
    out[...] = jnp.dot(pn, wn[...], preferred_element_type=jnp.float32) + \
        jnp.where(msk, bn[...], 0.0)


def kernel(feat, segment_ids, Wg, bg, Wn, bn):
    seg32 = segment_ids.astype(jnp.int32)
    seg3 = seg32.reshape(NB, 1, B)
    bg2 = bg.reshape(1, 1)
    bn2 = bn.reshape(1, D)

    gate, m, cnt = pl.pallas_call(
        _gate_kernel,
        grid=(NB,),
        in_specs=[
            pl.BlockSpec((B, D), lambda i: (i, 0)),
            pl.BlockSpec((1, 1, B), lambda i: (i, 0, 0)),
            pl.BlockSpec((D, 1), lambda i: (0, 0)),
            pl.BlockSpec((1, 1), lambda i: (0, 0)),
        ],
        out_specs=[
            pl.BlockSpec((B, 1), lambda i: (i, 0)),
            pl.BlockSpec((1, G), lambda i: (0, 0)),
            pl.BlockSpec((1, G), lambda i: (0, 0)),
        ],
        out_shape=[
            jax.ShapeDtypeStruct((N, 1), jnp.float32),
            jax.ShapeDtypeStruct((1, G), jnp.float32),
            jax.ShapeDtypeStruct((1, G), jnp.float32),
        ],
    )(feat, seg3, Wg, bg2)

    p, den = pl.pallas_call(
        _pexp_kernel,
        grid=(NB,),
        in_specs=[
            pl.BlockSpec((B, 1), lambda i: (i, 0)),
            pl.BlockSpec((1, 1, B), lambda i: (i, 0, 0)),
            pl.BlockSpec((1, G), lambda i: (0, 0)),
        ],
        out_specs=[
            pl.BlockSpec((B, 1), lambda i: (i, 0)),
            pl.BlockSpec((1, G), lambda i: (0, 0)),
        ],
        out_shape=[
            jax.ShapeDtypeStruct((N, 1), jnp.float32),
            jax.ShapeDtypeStruct((1, G), jnp.float32),
        ],
    )(gate, seg3, m)

    cnti = cnt[0].astype(jnp.int32)
    starts = jnp.concatenate(
        [jnp.zeros((1,), jnp.int32), jnp.cumsum(cnti, dtype=jnp.int32)])
    starts = jnp.pad(starts, (0, 15), constant_values=N)

    sc_pool = pl.kernel(
        _sc_pool_kernel,
        out_type=jax.ShapeDtypeStruct((G * D,), jnp.float32),
        mesh=plsc.VectorSubcoreMesh(core_axis_name="c", subcore_axis_name="s"),
        scratch_types=[
            pltpu.VMEM((16,), jnp.int32),
            pltpu.VMEM((RB, D), jnp.float32),
            pltpu.VMEM((RB,), jnp.float32),
            pltpu.VMEM((RB,), jnp.int32),
            pltpu.VMEM((ACC,), jnp.float32),
        ],
        compiler_params=pltpu.CompilerParams(needs_layout_passes=False),
    )
    pooled = sc_pool(feat, p.reshape(N), seg32, starts).reshape(G, D)

    out = pl.pallas_call(
        _final_kernel,
        in_specs=[
            pl.BlockSpec((G, D), lambda: (0, 0)),
            pl.BlockSpec((1, G), lambda: (0, 0)),
            pl.BlockSpec((1, G), lambda: (0, 0)),
            pl.BlockSpec((D, D), lambda: (0, 0)),
            pl.BlockSpec((1, D), lambda: (0, 0)),
        ],
        out_specs=pl.BlockSpec((G, D), lambda: (0, 0)),
        out_shape=jax.ShapeDtypeStruct((G, D), jnp.float32),
    )(pooled, den, cnt, Wn, bn2)

    return out


# speedup vs baseline: 2.6291x; 2.6291x over previous
import functools
import jax
import jax.numpy as jnp
from jax import lax
from jax.experimental import pallas as pl
from jax.experimental.pallas import tpu as pltpu
from jax.experimental.pallas import tpu_sc as plsc

N = 50000
D = 512
G = 256
B = 2000
NB = N // B

NW = 32
SEG_PER_W = G // NW
RB = 80
NBUF = N // RB
SB = RB // 16
FB = RB * D
ACC = SEG_PER_W * D
NC16 = D // 16

_NEG = -1e30


def _gate_kernel(feat, seg, wg, bg, gate, m_out, cnt_out):
    i = pl.program_id(0)

    @pl.when(i == 0)
    def _():
        m_out[...] = jnp.full_like(m_out, _NEG)
        cnt_out[...] = jnp.zeros_like(cnt_out)

    x = feat[...]
    g = jnp.dot(x, wg[...], preferred_element_type=jnp.float32) + bg[0, 0]
    gate[...] = g
    s = seg[0, 0, :]
    ids = lax.broadcasted_iota(jnp.int32, (B, G), 1)
    oh = s[:, None] == ids
    lm = jnp.max(jnp.where(oh, g, _NEG), axis=0)
    m_out[0, :] = jnp.maximum(m_out[0, :], lm)
    cnt_out[0, :] = cnt_out[0, :] + jnp.sum(oh.astype(jnp.float32), axis=0)


def _pexp_kernel(gate, seg, m, p_out, den_out):
    i = pl.program_id(0)

    @pl.when(i == 0)
    def _():
        den_out[...] = jnp.zeros_like(den_out)

    g = gate[...]
    s = seg[0, 0, :]
    ids = lax.broadcasted_iota(jnp.int32, (B, G), 1)
    oh = s[:, None] == ids
    m_sel = jnp.sum(jnp.where(oh, m[0, :][None, :], 0.0), axis=1)
    pv = jnp.exp(g[:, 0] - m_sel)
    p_out[...] = pv[:, None]
    den_out[0, :] = den_out[0, :] + jnp.sum(jnp.where(oh, pv[:, None], 0.0), axis=0)


def _pool_kernel(feat, seg, p, pooled):
    i = pl.program_id(0)

    @pl.when(i == 0)
    def _():
        pooled[...] = jnp.zeros_like(pooled)

    x = feat[...]
    w = x * p[...]
    s = seg[0, 0, :]
    ids = lax.broadcasted_iota(jnp.int32, (B, G), 1)
    oh = (s[:, None] == ids).astype(jnp.float32)
    pooled[...] = pooled[...] + lax.dot_general(
        oh, w, dimension_numbers=(((0,), (0,)), ((), ())),
        preferred_element_type=jnp.float32)


def _sc_pool_kernel(feat_hbm, p_hbm, seg_hbm, starts_hbm, out_hbm,
                    sv, fbuf, pbuf, sbuf, acc, sem0, sem1):
    c = lax.axis_index("c")
    s = lax.axis_index("s")
    w = s * 2 + c
    lo8 = w * SEG_PER_W
    pltpu.sync_copy(starts_hbm.at[pl.ds(lo8, 16)], sv)
    svv = sv[...]
    lo = svv[0]
    hi = svv[SEG_PER_W]
    b0 = lo // RB
    b1 = (hi + RB - 1) // RB

    zeros16 = jnp.zeros((16,), jnp.float32)
    for k in range(ACC // 16):
        acc[pl.ds(16 * k, 16)] = zeros16

    def _issue(i, par, sem):
        pltpu.async_copy(feat_hbm.at[pl.ds(i * FB, FB)],
                         fbuf.at[pl.ds(par * FB, FB)], sem)
        pltpu.async_copy(p_hbm.at[pl.ds(i * RB, RB)],
                         pbuf.at[pl.ds(par * RB, RB)], sem)
        pltpu.async_copy(seg_hbm.at[pl.ds(i * RB, RB)],
                         sbuf.at[pl.ds(par * RB, RB)], sem)

    def _drain(i, par, sem):
        pltpu.make_async_copy(feat_hbm.at[pl.ds(i * FB, FB)],
                              fbuf.at[pl.ds(par * FB, FB)], sem).wait()
        pltpu.make_async_copy(p_hbm.at[pl.ds(i * RB, RB)],
                              pbuf.at[pl.ds(par * RB, RB)], sem).wait()
        pltpu.make_async_copy(seg_hbm.at[pl.ds(i * RB, RB)],
                              sbuf.at[pl.ds(par * RB, RB)], sem).wait()

    @pl.when(b0 < b1)
    def _():
        _issue(b0, 0, sem0)

    def buf_body(i, carry):
        run = carry[0]
        accs = list(carry[1:])
        par = lax.rem(i - b0, 2)
        nxt = i + 1

        @pl.when(nxt < b1)
        def _():
            @pl.when(par == 0)
            def _():
                _issue(nxt, 1, sem1)

            @pl.when(par == 1)
            def _():
                _issue(nxt, 0, sem0)

        @pl.when(par == 0)
        def _():
            _drain(i, 0, sem0)

        @pl.when(par == 1)
        def _():
            _drain(i, 1, sem1)

        def sb_body(sb, carry2):
            run2 = carry2[0]
            a = list(carry2[1:])
            rbase = par * RB + sb * 16
            svec = sbuf[pl.ds(rbase, 16)]
            pvec = pbuf[pl.ds(rbase, 16)]
            fbase = par * FB + sb * 16 * D
            for r in range(16):
                s_r = svec[r]
                p_r = pvec[r]
                ok = (s_r >= lo8) & (s_r < lo8 + SEG_PER_W)
                eff = jnp.where(ok, s_r, -1)
                changed = eff != run2
                do_flush = changed & (run2 >= 0)
                abase = (run2 - lo8) * D

                @pl.when(do_flush)
                def _():
                    for j in range(NC16):
                        acc[pl.ds(abase + 16 * j, 16)] = a[j]

                contrib = jnp.where(ok, p_r, 0.0)
                zf = changed & ok
                row0 = fbase + r * D
                a = [jnp.where(zf, 0.0, a[j]) +
                     contrib * fbuf[pl.ds(row0 + 16 * j, 16)]
                     for j in range(NC16)]
                run2 = eff
            return tuple([run2] + a)

        carry_out = lax.fori_loop(0, SB, sb_body, tuple([run] + accs))
        return carry_out

    init = tuple([jnp.int32(-1)] + [zeros16] * NC16)
    fin = lax.fori_loop(b0, b1, buf_body, init)
    run_f = fin[0]
    abase_f = (run_f - lo8) * D

    @pl.when(run_f >= 0)
    def _():
        for j in range(NC16):
            acc[pl.ds(abase_f + 16 * j, 16)] = fin[1 + j]

    pltpu.sync_copy(acc, out_hbm.at[pl.ds(w * ACC, ACC)])


def _final_kernel(pooled, den, cnt, wn, bn, out):
    d = den[0, :][:, None]
    msk = cnt[0, :][:, None] > 0.5
    inv = jnp.where(d > 0, 1.0 / jnp.where(d > 0, d, 1.0), 0.0)
    pn = pooled[...] * inv
    out[...] = jnp.dot(pn, wn[...], preferred_element_type=jnp.float32) + \
        jnp.where(msk, bn[...], 0.0)


def kernel(feat, segment_ids, Wg, bg, Wn, bn):
    seg32 = segment_ids.astype(jnp.int32)
    seg3 = seg32.reshape(NB, 1, B)
    bg2 = bg.reshape(1, 1)
    bn2 = bn.reshape(1, D)

    gate, m, cnt = pl.pallas_call(
        _gate_kernel,
        grid=(NB,),
        in_specs=[
            pl.BlockSpec((B, D), lambda i: (i, 0)),
            pl.BlockSpec((1, 1, B), lambda i: (i, 0, 0)),
            pl.BlockSpec((D, 1), lambda i: (0, 0)),
            pl.BlockSpec((1, 1), lambda i: (0, 0)),
        ],
        out_specs=[
            pl.BlockSpec((B, 1), lambda i: (i, 0)),
            pl.BlockSpec((1, G), lambda i: (0, 0)),
            pl.BlockSpec((1, G), lambda i: (0, 0)),
        ],
        out_shape=[
            jax.ShapeDtypeStruct((N, 1), jnp.float32),
            jax.ShapeDtypeStruct((1, G), jnp.float32),
            jax.ShapeDtypeStruct((1, G), jnp.float32),
        ],
    )(feat, seg3, Wg, bg2)

    p, den = pl.pallas_call(
        _pexp_kernel,
        grid=(NB,),
        in_specs=[
            pl.BlockSpec((B, 1), lambda i: (i, 0)),
            pl.BlockSpec((1, 1, B), lambda i: (i, 0, 0)),
            pl.BlockSpec((1, G), lambda i: (0, 0)),
        ],
        out_specs=[
            pl.BlockSpec((B, 1), lambda i: (i, 0)),
            pl.BlockSpec((1, G), lambda i: (0, 0)),
        ],
        out_shape=[
            jax.ShapeDtypeStruct((N, 1), jnp.float32),
            jax.ShapeDtypeStruct((1, G), jnp.float32),
        ],
    )(gate, seg3, m)

    cnti = cnt[0].astype(jnp.int32)
    starts = jnp.concatenate(
        [jnp.zeros((1,), jnp.int32), jnp.cumsum(cnti, dtype=jnp.int32)])
    starts = jnp.pad(starts, (0, 15), constant_values=N)

    sc_pool = pl.kernel(
        _sc_pool_kernel,
        out_type=jax.ShapeDtypeStruct((G * D,), jnp.float32),
        mesh=plsc.VectorSubcoreMesh(core_axis_name="c", subcore_axis_name="s"),
        scratch_types=[
            pltpu.VMEM((16,), jnp.int32),
            pltpu.VMEM((2 * FB,), jnp.float32),
            pltpu.VMEM((2 * RB,), jnp.float32),
            pltpu.VMEM((2 * RB,), jnp.int32),
            pltpu.VMEM((ACC,), jnp.float32),
            pltpu.SemaphoreType.DMA,
            pltpu.SemaphoreType.DMA,
        ],
        compiler_params=pltpu.CompilerParams(needs_layout_passes=False),
    )
    pooled = sc_pool(feat.reshape(N * D), p.reshape(N), seg32,
                     starts).reshape(G, D)

    out = pl.pallas_call(
        _final_kernel,
        in_specs=[
            pl.BlockSpec((G, D), lambda: (0, 0)),
            pl.BlockSpec((1, G), lambda: (0, 0)),
            pl.BlockSpec((1, G), lambda: (0, 0)),
            pl.BlockSpec((D, D), lambda: (0, 0)),
            pl.BlockSpec((1, D), lambda: (0, 0)),
        ],
        out_specs=pl.BlockSpec((G, D), lambda: (0, 0)),
        out_shape=jax.ShapeDtypeStruct((G, D), jnp.float32),
    )(pooled, den, cnt, Wn, bn2)

    return out


# speedup vs baseline: 3.4503x; 1.3124x over previous
import functools
import jax
import jax.numpy as jnp
from jax import lax
from jax.experimental import pallas as pl
from jax.experimental.pallas import tpu as pltpu
from jax.experimental.pallas import tpu_sc as plsc

N = 50000
D = 512
G = 256
B = 2000
NB = N // B

NW = 32
SEG_PER_W = G // NW
RB = 80
NBUF = N // RB
SB = RB // 16
FB = RB * D
ACC = SEG_PER_W * D
NC16 = D // 16

_NEG = -1e30


def _gate_kernel(feat, seg, wg, bg, gate, m_out, cnt_out):
    i = pl.program_id(0)

    @pl.when(i == 0)
    def _():
        m_out[...] = jnp.full_like(m_out, _NEG)
        cnt_out[...] = jnp.zeros_like(cnt_out)

    x = feat[...]
    g = jnp.dot(x, wg[...], preferred_element_type=jnp.float32) + bg[0, 0]
    gate[...] = g
    s = seg[0, 0, :]
    ids = lax.broadcasted_iota(jnp.int32, (B, G), 1)
    oh = s[:, None] == ids
    lm = jnp.max(jnp.where(oh, g, _NEG), axis=0)
    m_out[0, :] = jnp.maximum(m_out[0, :], lm)
    cnt_out[0, :] = cnt_out[0, :] + jnp.sum(oh.astype(jnp.float32), axis=0)


def _pexp_kernel(gate, seg, m, p_out, den_out):
    i = pl.program_id(0)

    @pl.when(i == 0)
    def _():
        den_out[...] = jnp.zeros_like(den_out)

    g = gate[...]
    s = seg[0, 0, :]
    ids = lax.broadcasted_iota(jnp.int32, (B, G), 1)
    oh = s[:, None] == ids
    m_sel = jnp.sum(jnp.where(oh, m[0, :][None, :], 0.0), axis=1)
    pv = jnp.exp(g[:, 0] - m_sel)
    p_out[...] = pv[:, None]
    den_out[0, :] = den_out[0, :] + jnp.sum(jnp.where(oh, pv[:, None], 0.0), axis=0)


def _pool_kernel(feat, seg, p, pooled):
    i = pl.program_id(0)

    @pl.when(i == 0)
    def _():
        pooled[...] = jnp.zeros_like(pooled)

    x = feat[...]
    w = x * p[...]
    s = seg[0, 0, :]
    ids = lax.broadcasted_iota(jnp.int32, (B, G), 1)
    oh = (s[:, None] == ids).astype(jnp.float32)
    pooled[...] = pooled[...] + lax.dot_general(
        oh, w, dimension_numbers=(((0,), (0,)), ((), ())),
        preferred_element_type=jnp.float32)


def _sc_pool_kernel(feat_hbm, p_hbm, seg_hbm, starts_hbm, out_hbm,
                    sv, fbuf, pbuf, sbuf, acc, sem0, sem1):
    c = lax.axis_index("c")
    s = lax.axis_index("s")
    w = s * 2 + c
    lo8 = w * SEG_PER_W
    pltpu.sync_copy(starts_hbm.at[pl.ds(lo8, 16)], sv)
    svv = sv[...]
    lo = svv[0]
    hi = svv[SEG_PER_W]
    b0 = lo // RB
    b1 = (hi + RB - 1) // RB

    zeros16 = jnp.zeros((16,), jnp.float32)
    for k in range(ACC // 16):
        acc[pl.ds(16 * k, 16)] = zeros16

    def _issue(i, par, sem):
        pltpu.async_copy(feat_hbm.at[pl.ds(i * RB, RB), :],
                         fbuf.at[pl.ds(par * RB, RB), :], sem)
        pltpu.async_copy(p_hbm.at[pl.ds(i * RB, RB)],
                         pbuf.at[pl.ds(par * RB, RB)], sem)
        pltpu.async_copy(seg_hbm.at[pl.ds(i * RB, RB)],
                         sbuf.at[pl.ds(par * RB, RB)], sem)

    def _drain(i, par, sem):
        pltpu.make_async_copy(feat_hbm.at[pl.ds(i * RB, RB), :],
                              fbuf.at[pl.ds(par * RB, RB), :], sem).wait()
        pltpu.make_async_copy(p_hbm.at[pl.ds(i * RB, RB)],
                              pbuf.at[pl.ds(par * RB, RB)], sem).wait()
        pltpu.make_async_copy(seg_hbm.at[pl.ds(i * RB, RB)],
                              sbuf.at[pl.ds(par * RB, RB)], sem).wait()

    @pl.when(b0 < b1)
    def _():
        _issue(b0, 0, sem0)

    def buf_body(i, carry):
        run = carry[0]
        accs = list(carry[1:])
        par = lax.rem(i - b0, 2)
        nxt = i + 1

        @pl.when(nxt < b1)
        def _():
            @pl.when(par == 0)
            def _():
                _issue(nxt, 1, sem1)

            @pl.when(par == 1)
            def _():
                _issue(nxt, 0, sem0)

        @pl.when(par == 0)
        def _():
            _drain(i, 0, sem0)

        @pl.when(par == 1)
        def _():
            _drain(i, 1, sem1)

        def sb_body(sb, carry2):
            run2 = carry2[0]
            a = list(carry2[1:])
            rbase = par * RB + sb * 16
            svec = sbuf[pl.ds(rbase, 16)]
            pvec = pbuf[pl.ds(rbase, 16)]
            for r in range(16):
                s_r = svec[r]
                p_r = pvec[r]
                ok = (s_r >= lo8) & (s_r < lo8 + SEG_PER_W)
                eff = jnp.where(ok, s_r, -1)
                changed = eff != run2
                do_flush = changed & (run2 >= 0)
                abase = (run2 - lo8) * D

                @pl.when(do_flush)
                def _():
                    for j in range(NC16):
                        acc[pl.ds(abase + 16 * j, 16)] = a[j]

                contrib = jnp.where(ok, p_r, 0.0)
                zf = changed & ok
                row = rbase + r
                a = [jnp.where(zf, 0.0, a[j]) +
                     contrib * fbuf[row, pl.ds(16 * j, 16)]
                     for j in range(NC16)]
                run2 = eff
            return tuple([run2] + a)

        carry_out = lax.fori_loop(0, SB, sb_body, tuple([run] + accs))
        return carry_out

    init = tuple([jnp.int32(-1)] + [zeros16] * NC16)
    fin = lax.fori_loop(b0, b1, buf_body, init)
    run_f = fin[0]
    abase_f = (run_f - lo8) * D

    @pl.when(run_f >= 0)
    def _():
        for j in range(NC16):
            acc[pl.ds(abase_f + 16 * j, 16)] = fin[1 + j]

    pltpu.sync_copy(acc, out_hbm.at[pl.ds(w * ACC, ACC)])


def _final_kernel(pooled, den, cnt, wn, bn, out):
    d = den[0, :][:, None]
    msk = cnt[0, :][:, None] > 0.5
    inv = jnp.where(d > 0, 1.0 / jnp.where(d > 0, d, 1.0), 0.0)
    pn = pooled[...] * inv
    out[...] = jnp.dot(pn, wn[...], preferred_element_type=jnp.float32) + \
        jnp.where(msk, bn[...], 0.0)


def kernel(feat, segment_ids, Wg, bg, Wn, bn):
    seg32 = segment_ids.astype(jnp.int32)
    seg3 = seg32.reshape(NB, 1, B)
    bg2 = bg.reshape(1, 1)
    bn2 = bn.reshape(1, D)

    gate, m, cnt = pl.pallas_call(
        _gate_kernel,
        grid=(NB,),
        in_specs=[
            pl.BlockSpec((B, D), lambda i: (i, 0)),
            pl.BlockSpec((1, 1, B), lambda i: (i, 0, 0)),
            pl.BlockSpec((D, 1), lambda i: (0, 0)),
            pl.BlockSpec((1, 1), lambda i: (0, 0)),
        ],
        out_specs=[
            pl.BlockSpec((B, 1), lambda i: (i, 0)),
            pl.BlockSpec((1, G), lambda i: (0, 0)),
            pl.BlockSpec((1, G), lambda i: (0, 0)),
        ],
        out_shape=[
            jax.ShapeDtypeStruct((N, 1), jnp.float32),
            jax.ShapeDtypeStruct((1, G), jnp.float32),
            jax.ShapeDtypeStruct((1, G), jnp.float32),
        ],
    )(feat, seg3, Wg, bg2)

    p, den = pl.pallas_call(
        _pexp_kernel,
        grid=(NB,),
        in_specs=[
            pl.BlockSpec((B, 1), lambda i: (i, 0)),
            pl.BlockSpec((1, 1, B), lambda i: (i, 0, 0)),
            pl.BlockSpec((1, G), lambda i: (0, 0)),
        ],
        out_specs=[
            pl.BlockSpec((B, 1), lambda i: (i, 0)),
            pl.BlockSpec((1, G), lambda i: (0, 0)),
        ],
        out_shape=[
            jax.ShapeDtypeStruct((N, 1), jnp.float32),
            jax.ShapeDtypeStruct((1, G), jnp.float32),
        ],
    )(gate, seg3, m)

    cnti = cnt[0].astype(jnp.int32)
    starts = jnp.concatenate(
        [jnp.zeros((1,), jnp.int32), jnp.cumsum(cnti, dtype=jnp.int32)])
    starts = jnp.pad(starts, (0, 15), constant_values=N)

    sc_pool = pl.kernel(
        _sc_pool_kernel,
        out_type=jax.ShapeDtypeStruct((G * D,), jnp.float32),
        mesh=plsc.VectorSubcoreMesh(core_axis_name="c", subcore_axis_name="s"),
        scratch_types=[
            pltpu.VMEM((16,), jnp.int32),
            pltpu.VMEM((2 * RB, D), jnp.float32),
            pltpu.VMEM((2 * RB,), jnp.float32),
            pltpu.VMEM((2 * RB,), jnp.int32),
            pltpu.VMEM((ACC,), jnp.float32),
            pltpu.SemaphoreType.DMA,
            pltpu.SemaphoreType.DMA,
        ],
        compiler_params=pltpu.CompilerParams(needs_layout_passes=False),
    )
    pooled = sc_pool(feat, p.reshape(N), seg32, starts).reshape(G, D)

    out = pl.pallas_call(
        _final_kernel,
        in_specs=[
            pl.BlockSpec((G, D), lambda: (0, 0)),
            pl.BlockSpec((1, G), lambda: (0, 0)),
            pl.BlockSpec((1, G), lambda: (0, 0)),
            pl.BlockSpec((D, D), lambda: (0, 0)),
            pl.BlockSpec((1, D), lambda: (0, 0)),
        ],
        out_specs=pl.BlockSpec((G, D), lambda: (0, 0)),
        out_shape=jax.ShapeDtypeStruct((G, D), jnp.float32),
    )(pooled, den, cnt, Wn, bn2)

    return out
